# baseline (speedup 1.0000x reference)
import functools

import jax
import jax.numpy as jnp
from jax import lax
from jax.experimental import pallas as pl
from jax.experimental.pallas import tpu as pltpu
from jax.experimental.pallas import tpu_sc as plsc

_NC = 2
_NS = 16
_CH = 128


@functools.lru_cache(maxsize=None)
def _build_segsum(Np, H, per):
    NW = _NC * _NS
    rpt = Np // _NS
    SEG = per
    while 16 * (2 * SEG * _CH + 2 * _CH * H) + Np * H > 2_000_000:
        SEG //= 2
    assert per % SEG == 0 and SEG % 2 == 0
    nstage = per // SEG
    mesh = plsc.VectorSubcoreMesh(core_axis_name="c", subcore_axis_name="s")

    @functools.partial(
        pl.kernel,
        mesh=mesh,
        out_type=jax.ShapeDtypeStruct((_NC, Np, H), jnp.float32),
        scratch_types=[
            pltpu.VMEM((SEG, _CH), jnp.int32),
            pltpu.VMEM((SEG, _CH), jnp.int32),
            pltpu.VMEM((_CH, H), jnp.float32),
            pltpu.VMEM((_CH, H), jnp.float32),
            pltpu.VMEM_SHARED((Np, H), jnp.float32),
            pltpu.SemaphoreType.DMA,
            pltpu.SemaphoreType.DMA,
        ],
    )
    def segsum(h_hbm, src2_hbm, dst2_hbm, zeros_hbm, out_hbm,
               sidx_all, didx_all, rows0, rows1, acc, sem0, sem1):
        cid = lax.axis_index("c")
        sid = lax.axis_index("s")
        wid = sid * _NC + cid
        r0 = sid * rpt
        pltpu.sync_copy(zeros_hbm.at[pl.ds(r0, rpt)], acc.at[pl.ds(r0, rpt)])
        c0 = wid * per
        plsc.subcore_barrier()

        def gstart(j, buf, sem):
            pltpu.make_async_copy(h_hbm.at[sidx_all.at[j]], buf, sem).start()

        def gwait(j, buf, sem):
            pltpu.make_async_copy(h_hbm.at[sidx_all.at[j]], buf, sem).wait()

        def scatter(j, buf):
            pltpu.sync_copy(buf, acc.at[didx_all.at[j]], add=True)

        def body(k, carry):
            j = 2 * k
            gstart(j + 1, rows1, sem1)
            gwait(j, rows0, sem0)
            scatter(j, rows0)

            @pl.when(j + 2 < SEG)
            def _():
                gstart(j + 2, rows0, sem0)

            gwait(j + 1, rows1, sem1)
            scatter(j + 1, rows1)
            return carry

        for s in range(nstage):
            pltpu.sync_copy(src2_hbm.at[pl.ds(c0 + s * SEG, SEG)], sidx_all)
            pltpu.sync_copy(dst2_hbm.at[pl.ds(c0 + s * SEG, SEG)], didx_all)
            gstart(0, rows0, sem0)
            lax.fori_loop(0, SEG // 2, body, 0)

        plsc.subcore_barrier()
        pltpu.sync_copy(acc.at[pl.ds(r0, rpt)],
                        out_hbm.at[cid, pl.ds(r0, rpt)])

    return segsum


@functools.lru_cache(maxsize=None)
def _build_count(Np, per, W=128):
    NW = _NC * _NS
    rpt = Np // _NS
    mesh = plsc.VectorSubcoreMesh(core_axis_name="c", subcore_axis_name="s")

    @functools.partial(
        pl.kernel,
        mesh=mesh,
        out_type=jax.ShapeDtypeStruct((_NC, Np, W), jnp.float32),
        scratch_types=[
            pltpu.VMEM((per, _CH), jnp.int32),
            pltpu.VMEM((_CH, W), jnp.float32),
            pltpu.VMEM_SHARED((Np, W), jnp.float32),
        ],
    )
    def count(dst2_hbm, ones_hbm, zerosw_hbm, out_hbm, didx_all, ones_v, cacc):
        cid = lax.axis_index("c")
        sid = lax.axis_index("s")
        wid = sid * _NC + cid
        r0 = sid * rpt
        pltpu.sync_copy(ones_hbm, ones_v)
        pltpu.sync_copy(zerosw_hbm.at[pl.ds(r0, rpt)], cacc.at[pl.ds(r0, rpt)])
        c0 = wid * per
        pltpu.sync_copy(dst2_hbm.at[pl.ds(c0, per)], didx_all)
        plsc.subcore_barrier()

        def body(j, carry):
            pltpu.sync_copy(ones_v, cacc.at[didx_all.at[j]], add=True)
            return carry

        lax.fori_loop(0, per, body, 0)

        plsc.subcore_barrier()
        pltpu.sync_copy(cacc.at[pl.ds(r0, rpt)],
                        out_hbm.at[cid, pl.ds(r0, rpt)])

    return count


def _fuse(p, h, cnt, Wl_i, Wr_i, b_i, relu, nb=8):
    Np, H = h.shape
    BR = Np // nb

    def body(p_ref, h_ref, cnt_ref, wl_ref, wr_ref, b_ref, o_ref):
        inv = 1.0 / jnp.maximum(cnt_ref[...], 1.0)
        agg = (p_ref[0] + p_ref[1]) * inv
        acc = jnp.dot(agg, wl_ref[...], preferred_element_type=jnp.float32)
        acc = acc + jnp.dot(h_ref[...], wr_ref[...],
                            preferred_element_type=jnp.float32)
        acc = acc + b_ref[...]
        if relu:
            acc = jnp.maximum(acc, 0.0)
        o_ref[...] = acc

    return pl.pallas_call(
        body,
        grid=(nb,),
        in_specs=[
            pl.BlockSpec((2, BR, H), lambda i: (0, i, 0)),
            pl.BlockSpec((BR, H), lambda i: (i, 0)),
            pl.BlockSpec((BR, 1), lambda i: (i, 0)),
            pl.BlockSpec((H, H), lambda i: (0, 0)),
            pl.BlockSpec((H, H), lambda i: (0, 0)),
            pl.BlockSpec((1, H), lambda i: (0, 0)),
        ],
        out_specs=pl.BlockSpec((BR, H), lambda i: (i, 0)),
        out_shape=jax.ShapeDtypeStruct((Np, H), jnp.float32),
    )(p, h, cnt, Wl_i, Wr_i, b_i.reshape(1, H))


def kernel(x, edge_index, Wl, Wr, b):
    N, D = x.shape
    E = edge_index.shape[1]
    L = Wl.shape[0]
    NW = _NC * _NS

    Np = (N // 128 + 1) * 128
    nchunks = -(-E // _CH)
    per = -(-nchunks // NW)
    per = per + (per % 2)
    Ep = per * NW * _CH

    src = jnp.full((Ep,), N, jnp.int32).at[:E].set(edge_index[0])
    dst = jnp.full((Ep,), N, jnp.int32).at[:E].set(edge_index[1])
    src2 = src.reshape(Ep // _CH, _CH)
    dst2 = dst.reshape(Ep // _CH, _CH)
    hp = jnp.zeros((Np, D), jnp.float32).at[:N].set(x)
    zeros = jnp.zeros((Np, D), jnp.float32)

    segsum = _build_segsum(Np, D, per)
    cparts = segsum(jnp.ones((Np, D), jnp.float32), src2, dst2, zeros)
    cnt = (cparts[0, :, :1] + cparts[1, :, :1])
    h = hp
    for i in range(L):
        p = segsum(h, src2, dst2, zeros)
        h = _fuse(p, h, cnt, Wl[i], Wr[i], b[i], relu=(i < L - 1))
    return h[:N]


# speedup vs baseline: 1.0676x; 1.0676x over previous
import functools

import jax
import jax.numpy as jnp
from jax import lax
from jax.experimental import pallas as pl
from jax.experimental.pallas import tpu as pltpu
from jax.experimental.pallas import tpu_sc as plsc

_NC = 2
_NS = 16
_CH = 128


@functools.lru_cache(maxsize=None)
def _build_segsum(Np, H, per):
    NW = _NC * _NS
    rpt = Np // _NS
    mesh = plsc.VectorSubcoreMesh(core_axis_name="c", subcore_axis_name="s")

    @functools.partial(
        pl.kernel,
        mesh=mesh,
        out_type=jax.ShapeDtypeStruct((_NC, Np, H), jnp.float32),
        scratch_types=[
            pltpu.VMEM((_CH,), jnp.int32),
            pltpu.VMEM((_CH,), jnp.int32),
            pltpu.VMEM((_CH,), jnp.int32),
            pltpu.VMEM((_CH,), jnp.int32),
            pltpu.VMEM((_CH, H), jnp.float32),
            pltpu.VMEM((_CH, H), jnp.float32),
            pltpu.VMEM_SHARED((Np, H), jnp.float32),
            pltpu.SemaphoreType.DMA,
            pltpu.SemaphoreType.DMA,
        ],
    )
    def segsum(h_hbm, src2_hbm, dst2_hbm, zeros_hbm, out_hbm,
               sidx0, sidx1, didx0, didx1, rows0, rows1, acc, sem0, sem1):
        cid = lax.axis_index("c")
        sid = lax.axis_index("s")
        wid = sid * _NC + cid
        r0 = sid * rpt
        pltpu.sync_copy(zeros_hbm.at[pl.ds(r0, rpt)], acc.at[pl.ds(r0, rpt)])
        c0 = wid * per
        plsc.subcore_barrier()

        def fetch_idx(row, sbuf, dbuf):
            pltpu.sync_copy(src2_hbm.at[row], sbuf)
            pltpu.sync_copy(dst2_hbm.at[row], dbuf)

        def gstart(sbuf, buf, sem):
            pltpu.make_async_copy(h_hbm.at[sbuf], buf, sem).start()

        def gwait(sbuf, buf, sem):
            pltpu.make_async_copy(h_hbm.at[sbuf], buf, sem).wait()

        def scatter(dbuf, buf):
            pltpu.sync_copy(buf, acc.at[dbuf], add=True)

        fetch_idx(c0, sidx0, didx0)
        gstart(sidx0, rows0, sem0)

        def body(k, carry):
            j = c0 + 2 * k
            fetch_idx(j + 1, sidx1, didx1)
            gstart(sidx1, rows1, sem1)
            gwait(sidx0, rows0, sem0)
            scatter(didx0, rows0)

            @pl.when(2 * k + 2 < per)
            def _():
                fetch_idx(j + 2, sidx0, didx0)
                gstart(sidx0, rows0, sem0)

            gwait(sidx1, rows1, sem1)
            scatter(didx1, rows1)
            return carry

        lax.fori_loop(0, per // 2, body, 0)

        plsc.subcore_barrier()
        pltpu.sync_copy(acc.at[pl.ds(r0, rpt)],
                        out_hbm.at[cid, pl.ds(r0, rpt)])

    return segsum


@functools.lru_cache(maxsize=None)
def _build_count(Np, per, W=128):
    NW = _NC * _NS
    rpt = Np // _NS
    mesh = plsc.VectorSubcoreMesh(core_axis_name="c", subcore_axis_name="s")

    @functools.partial(
        pl.kernel,
        mesh=mesh,
        out_type=jax.ShapeDtypeStruct((_NC, Np, W), jnp.float32),
        scratch_types=[
            pltpu.VMEM((_CH,), jnp.int32),
            pltpu.VMEM((_CH, W), jnp.float32),
            pltpu.VMEM_SHARED((Np, W), jnp.float32),
        ],
    )
    def count(dst2_hbm, ones_hbm, zerosw_hbm, out_hbm, didx, ones_v, cacc):
        cid = lax.axis_index("c")
        sid = lax.axis_index("s")
        wid = sid * _NC + cid
        r0 = sid * rpt
        pltpu.sync_copy(ones_hbm, ones_v)
        pltpu.sync_copy(zerosw_hbm.at[pl.ds(r0, rpt)], cacc.at[pl.ds(r0, rpt)])
        c0 = wid * per
        plsc.subcore_barrier()

        def body(j, carry):
            pltpu.sync_copy(dst2_hbm.at[c0 + j], didx)
            pltpu.sync_copy(ones_v, cacc.at[didx], add=True)
            return carry

        lax.fori_loop(0, per, body, 0)

        plsc.subcore_barrier()
        pltpu.sync_copy(cacc.at[pl.ds(r0, rpt)],
                        out_hbm.at[cid, pl.ds(r0, rpt)])

    return count


def _fuse(p, h, cnt, Wl_i, Wr_i, b_i, relu, nb=8):
    Np, H = h.shape
    BR = Np // nb

    def body(p_ref, h_ref, cnt_ref, wl_ref, wr_ref, b_ref, o_ref):
        inv = 1.0 / jnp.maximum(cnt_ref[...], 1.0)
        agg = (p_ref[0] + p_ref[1]) * inv
        acc = jnp.dot(agg, wl_ref[...], preferred_element_type=jnp.float32)
        acc = acc + jnp.dot(h_ref[...], wr_ref[...],
                            preferred_element_type=jnp.float32)
        acc = acc + b_ref[...]
        if relu:
            acc = jnp.maximum(acc, 0.0)
        o_ref[...] = acc

    return pl.pallas_call(
        body,
        grid=(nb,),
        in_specs=[
            pl.BlockSpec((2, BR, H), lambda i: (0, i, 0)),
            pl.BlockSpec((BR, H), lambda i: (i, 0)),
            pl.BlockSpec((BR, 1), lambda i: (i, 0)),
            pl.BlockSpec((H, H), lambda i: (0, 0)),
            pl.BlockSpec((H, H), lambda i: (0, 0)),
            pl.BlockSpec((1, H), lambda i: (0, 0)),
        ],
        out_specs=pl.BlockSpec((BR, H), lambda i: (i, 0)),
        out_shape=jax.ShapeDtypeStruct((Np, H), jnp.float32),
    )(p, h, cnt, Wl_i, Wr_i, b_i.reshape(1, H))


def kernel(x, edge_index, Wl, Wr, b):
    N, D = x.shape
    E = edge_index.shape[1]
    L = Wl.shape[0]
    NW = _NC * _NS

    Np = (N // 128 + 1) * 128
    nchunks = -(-E // _CH)
    per = -(-nchunks // NW)
    per = per + (per % 2)
    Ep = per * NW * _CH

    src = jnp.full((Ep,), N, jnp.int32).at[:E].set(edge_index[0])
    dst = jnp.full((Ep,), N, jnp.int32).at[:E].set(edge_index[1])
    src2 = src.reshape(Ep // _CH, _CH)
    dst2 = dst.reshape(Ep // _CH, _CH)
    hp = jnp.zeros((Np, D), jnp.float32).at[:N].set(x)
    zeros = jnp.zeros((Np, D), jnp.float32)
    onesw = jnp.ones((_CH, D), jnp.float32)

    cparts = _build_count(Np, per, D)(dst2, onesw, zeros)
    cnt = (cparts[0, :, :1] + cparts[1, :, :1])

    segsum = _build_segsum(Np, D, per)
    h = hp
    for i in range(L):
        p = segsum(h, src2, dst2, zeros)
        h = _fuse(p, h, cnt, Wl[i], Wr[i], b[i], relu=(i < L - 1))
    return h[:N]
